# Initial kernel scaffold; baseline (speedup 1.0000x reference)
import functools

import jax
import jax.numpy as jnp
from jax import lax
from jax.experimental import pallas as pl
from jax.experimental.pallas import tpu as pltpu
from jax.experimental.pallas import tpu_sc as plsc

EMB = 128
BATCH = 16384
NC = 2
NS = 16
NW = NC * NS
BPW = BATCH // NW
C = 128
NCH = BPW // C
LANES = 16

_mesh = plsc.VectorSubcoreMesh(core_axis_name="c", subcore_axis_name="s")


@functools.partial(
    pl.kernel,
    mesh=_mesh,
    out_type=jax.ShapeDtypeStruct((BATCH,), jnp.float32),
    scratch_types=[
        pltpu.VMEM((NCH, C), jnp.int32),
        pltpu.VMEM((NCH, C), jnp.int32),
        pltpu.VMEM((NCH, C), jnp.int32),
        pltpu.VMEM((C, EMB), jnp.float32),
        pltpu.VMEM((C, EMB), jnp.float32),
        pltpu.VMEM((C, EMB), jnp.float32),
        pltpu.VMEM((BPW,), jnp.float32),
        pltpu.SemaphoreType.DMA,
    ],
)
def _sc_diffs(ut, it, uix, pix, nix, out, uidx, pidx, nidx, ub, pb, nb, ov, sem):
    wid = lax.axis_index("s") * NC + lax.axis_index("c")
    pltpu.sync_copy(uix.at[wid], uidx)
    pltpu.sync_copy(pix.at[wid], pidx)
    pltpu.sync_copy(nix.at[wid], nidx)
    for j in range(NCH):
        cu = pltpu.async_copy(ut.at[uidx.at[j]], ub, sem)
        cp = pltpu.async_copy(it.at[pidx.at[j]], pb, sem)
        cn = pltpu.async_copy(it.at[nidx.at[j]], nb, sem)
        cu.wait()
        cp.wait()
        cn.wait()

        def row_body(r, _, j=j):
            acc = jnp.zeros((LANES,), jnp.float32)
            for k in range(EMB // LANES):
                u = ub[r, pl.ds(k * LANES, LANES)]
                p = pb[r, pl.ds(k * LANES, LANES)]
                n = nb[r, pl.ds(k * LANES, LANES)]
                acc = acc + u * (n - p)
            ov[j * C + r] = jnp.sum(acc)
            return 0

        lax.fori_loop(0, C, row_body, 0)
    pltpu.sync_copy(ov, out.at[pl.ds(pl.multiple_of(wid * BPW, 8), BPW)])


def _softplus_mean_body(x_ref, o_ref):
    x = x_ref[...]
    sp = jnp.maximum(x, 0.0) + jnp.log1p(jnp.exp(-jnp.abs(x)))
    o_ref[0, 0] = jnp.sum(sp) * (1.0 / BATCH)


_tc_reduce = pl.pallas_call(
    _softplus_mean_body,
    out_shape=jax.ShapeDtypeStruct((1, 1), jnp.float32),
    in_specs=[pl.BlockSpec(memory_space=pltpu.VMEM)],
    out_specs=pl.BlockSpec(memory_space=pltpu.SMEM),
)


def kernel(user_table, item_table, users, pos, neg):
    u = users.astype(jnp.int32).reshape(NW, NCH, C)
    p = pos.astype(jnp.int32).reshape(NW, NCH, C)
    n = neg.astype(jnp.int32).reshape(NW, NCH, C)
    diffs = _sc_diffs(user_table, item_table, u, p, n)
    loss = _tc_reduce(diffs.reshape(BATCH // EMB, EMB))[0, 0]
    return loss


# speedup vs baseline: 1.1939x; 1.1939x over previous
import functools

import jax
import jax.numpy as jnp
from jax import lax
from jax.experimental import pallas as pl
from jax.experimental.pallas import tpu as pltpu
from jax.experimental.pallas import tpu_sc as plsc

EMB = 128
BATCH = 16384
NC = 2
NS = 16
NW = NC * NS
BPW = BATCH // NW
C = 128
NCH = BPW // C
LANES = 16

_mesh = plsc.VectorSubcoreMesh(core_axis_name="c", subcore_axis_name="s")


@functools.partial(
    pl.kernel,
    mesh=_mesh,
    out_type=jax.ShapeDtypeStruct((BATCH, LANES), jnp.float32),
    scratch_types=[
        pltpu.VMEM((NCH, C), jnp.int32),
        pltpu.VMEM((NCH, C), jnp.int32),
        pltpu.VMEM((NCH, C), jnp.int32),
        pltpu.VMEM((C, EMB), jnp.float32),
        pltpu.VMEM((C, EMB), jnp.float32),
        pltpu.VMEM((C, EMB), jnp.float32),
        pltpu.VMEM((C, LANES), jnp.float32),
        pltpu.SemaphoreType.DMA,
    ],
)
def _sc_diffs(ut, it, uix, pix, nix, out, uidx, pidx, nidx, ub, pb, nb, ov, sem):
    wid = lax.axis_index("s") * NC + lax.axis_index("c")
    pltpu.sync_copy(uix.at[wid], uidx)
    pltpu.sync_copy(pix.at[wid], pidx)
    pltpu.sync_copy(nix.at[wid], nidx)
    for j in range(NCH):
        cu = pltpu.async_copy(ut.at[uidx.at[j]], ub, sem)
        cp = pltpu.async_copy(it.at[pidx.at[j]], pb, sem)
        cn = pltpu.async_copy(it.at[nidx.at[j]], nb, sem)
        cu.wait()
        cp.wait()
        cn.wait()

        def row_body(r, _):
            acc = jnp.zeros((LANES,), jnp.float32)
            for k in range(EMB // LANES):
                u = ub[r, pl.ds(k * LANES, LANES)]
                p = pb[r, pl.ds(k * LANES, LANES)]
                n = nb[r, pl.ds(k * LANES, LANES)]
                acc = acc + u * (n - p)
            ov[r, :] = acc
            return 0

        lax.fori_loop(0, C, row_body, 0)
        base = pl.multiple_of(wid * BPW + j * C, 8)
        pltpu.sync_copy(ov, out.at[pl.ds(base, C)])


def _softplus_mean_body(x_ref, o_ref):
    x = x_ref[...]
    d = jnp.sum(x.reshape(x.shape[0], x.shape[1] // LANES, LANES), axis=2)
    sp = jnp.maximum(d, 0.0) + jnp.log1p(jnp.exp(-jnp.abs(d)))
    o_ref[0, 0] = jnp.sum(sp) * (1.0 / BATCH)


_tc_reduce = pl.pallas_call(
    _softplus_mean_body,
    out_shape=jax.ShapeDtypeStruct((1, 1), jnp.float32),
    in_specs=[pl.BlockSpec(memory_space=pltpu.VMEM)],
    out_specs=pl.BlockSpec(memory_space=pltpu.SMEM),
)


def kernel(user_table, item_table, users, pos, neg):
    u = users.astype(jnp.int32).reshape(NW, NCH, C)
    p = pos.astype(jnp.int32).reshape(NW, NCH, C)
    n = neg.astype(jnp.int32).reshape(NW, NCH, C)
    partials = _sc_diffs(user_table, item_table, u, p, n)
    loss = _tc_reduce(partials.reshape(BATCH * LANES // EMB, EMB))[0, 0]
    return loss


# speedup vs baseline: 1.4505x; 1.2150x over previous
import functools

import jax
import jax.numpy as jnp
from jax import lax
from jax.experimental import pallas as pl
from jax.experimental.pallas import tpu as pltpu
from jax.experimental.pallas import tpu_sc as plsc

EMB = 128
BATCH = 16384
NC = 2
NS = 16
NW = NC * NS
BPW = BATCH // NW
C = 128
NCH = BPW // C
LANES = 16

_mesh = plsc.VectorSubcoreMesh(core_axis_name="c", subcore_axis_name="s")


OUT_ROWS = BATCH * LANES // EMB
ORPW = OUT_ROWS // NW
ORPC = ORPW // NCH


@functools.partial(
    pl.kernel,
    mesh=_mesh,
    out_type=jax.ShapeDtypeStruct((OUT_ROWS, EMB), jnp.float32),
    scratch_types=[
        pltpu.VMEM((NCH, C), jnp.int32),
        pltpu.VMEM((NCH, C), jnp.int32),
        pltpu.VMEM((NCH, C), jnp.int32),
        pltpu.VMEM((C, EMB), jnp.float32),
        pltpu.VMEM((C, EMB), jnp.float32),
        pltpu.VMEM((C, EMB), jnp.float32),
        pltpu.VMEM((C, EMB), jnp.float32),
        pltpu.VMEM((C, EMB), jnp.float32),
        pltpu.VMEM((C, EMB), jnp.float32),
        pltpu.VMEM((ORPC, EMB), jnp.float32),
        pltpu.SemaphoreType.DMA,
        pltpu.SemaphoreType.DMA,
    ],
)
def _sc_diffs(ut, it, uix, pix, nix, out, uidx, pidx, nidx,
              ub0, pb0, nb0, ub1, pb1, nb1, ov, sem0, sem1):
    wid = lax.axis_index("s") * NC + lax.axis_index("c")
    pltpu.sync_copy(uix.at[wid], uidx)
    pltpu.sync_copy(pix.at[wid], pidx)
    pltpu.sync_copy(nix.at[wid], nidx)
    bufs = ((ub0, pb0, nb0, sem0), (ub1, pb1, nb1, sem1))

    def start(j):
        ub, pb, nb, sem = bufs[j % 2]
        return (pltpu.async_copy(ut.at[uidx.at[j]], ub, sem),
                pltpu.async_copy(it.at[pidx.at[j]], pb, sem),
                pltpu.async_copy(it.at[nidx.at[j]], nb, sem))

    pend = start(0)
    for j in range(NCH):
        nxt = start(j + 1) if j + 1 < NCH else None
        for cpy in pend:
            cpy.wait()
        ub, pb, nb, _ = bufs[j % 2]

        def row8_body(o, _, ub=ub, pb=pb, nb=nb):
            for i in range(8):
                r = o * 8 + i
                acc = jnp.zeros((LANES,), jnp.float32)
                for k in range(EMB // LANES):
                    u = ub[r, pl.ds(k * LANES, LANES)]
                    p = pb[r, pl.ds(k * LANES, LANES)]
                    n = nb[r, pl.ds(k * LANES, LANES)]
                    acc = acc + u * (n - p)
                ov[o, pl.ds(i * LANES, LANES)] = acc
            return 0

        lax.fori_loop(0, ORPC, row8_body, 0)
        base = pl.multiple_of(wid * ORPW + j * ORPC, 8)
        pltpu.sync_copy(ov, out.at[pl.ds(base, ORPC)])
        pend = nxt


def _softplus_mean_body(x_ref, o_ref):
    x = x_ref[...]
    d = jnp.sum(x.reshape(x.shape[0], EMB // LANES, LANES), axis=2)
    sp = jnp.maximum(d, 0.0) + jnp.log1p(jnp.exp(-jnp.abs(d)))
    o_ref[0, 0] = jnp.sum(sp) * (1.0 / BATCH)


_tc_reduce = pl.pallas_call(
    _softplus_mean_body,
    out_shape=jax.ShapeDtypeStruct((1, 1), jnp.float32),
    in_specs=[pl.BlockSpec(memory_space=pltpu.VMEM)],
    out_specs=pl.BlockSpec(memory_space=pltpu.SMEM),
)


def kernel(user_table, item_table, users, pos, neg):
    u = users.astype(jnp.int32).reshape(NW, NCH, C)
    p = pos.astype(jnp.int32).reshape(NW, NCH, C)
    n = neg.astype(jnp.int32).reshape(NW, NCH, C)
    partials = _sc_diffs(user_table, item_table, u, p, n)
    loss = _tc_reduce(partials)[0, 0]
    return loss
